# Initial kernel scaffold; baseline (speedup 1.0000x reference)
import jax
import jax.numpy as jnp
from jax.experimental import pallas as pl


def kernel(text, embedding, fc_w, fc_b):
    raise NotImplementedError("write your pallas kernel here")


# speedup vs baseline: 1.4323x; 1.4323x over previous
import functools

import jax
import jax.numpy as jnp
from jax import lax
from jax.experimental import pallas as pl
from jax.experimental.pallas import tpu as pltpu, tpu_sc as plsc

_BATCH = 16384
_SEQ = 50
_D = 16
_NW = 32
_ROWS_PER_W = _BATCH // _NW
_GROUP_ROWS = 64
_GROUPS = _ROWS_PER_W // _GROUP_ROWS
_IDX_PER_GROUP = _GROUP_ROWS * _SEQ
_IDX_COLS = 128
_IDX_ROWS = _IDX_PER_GROUP // _IDX_COLS
_IDX2_ROWS = _BATCH * _SEQ // _IDX_COLS

_GATHER_DNUMS = lax.GatherDimensionNumbers(
    offset_dims=(), collapsed_slice_dims=(0,), start_index_map=(0,))


def _lane_shuffle(x, perm):
    return lax.gather(x, perm[:, None], _GATHER_DNUMS, (1,),
                      mode=lax.GatherScatterMode.PROMISE_IN_BOUNDS)


def _lane_sum(x, lanes):
    for sh in (8, 4, 2, 1):
        x = x + _lane_shuffle(x, lanes ^ sh)
    return x


def _sc_body(text_hbm, emb_hbm, w_hbm, b_hbm, out_hbm,
             idx_v, rows_v, w_v, b_v, out_v, sem):
    cid = lax.axis_index("c")
    sid = lax.axis_index("s")
    wid = cid * 16 + sid

    pltpu.sync_copy(w_hbm, w_v)
    pltpu.sync_copy(b_hbm, b_v)
    pltpu.sync_copy(
        text_hbm.at[pl.ds(wid * (_GROUPS * _IDX_ROWS), _GROUPS * _IDX_ROWS), :],
        idx_v)
    wv = w_v[...]
    bv = b_v[...]
    lanes = lax.iota(jnp.int32, 16)

    def group_body(g, carry):
        copies = []
        for j in range(_IDX_ROWS):
            copies.append(pltpu.async_copy(
                emb_hbm.at[idx_v.at[g * _IDX_ROWS + j]],
                rows_v.at[pl.ds(j * _IDX_COLS, _IDX_COLS)],
                sem))
        for c in copies:
            c.wait()

        def q_body(q, carry_q):
            def r_body(r16, ovec):
                base = (q * 16 + r16) * _SEQ

                def l_body(l, acc):
                    return acc + rows_v[base + l, :]

                acc = lax.fori_loop(0, _SEQ, l_body,
                                    jnp.zeros((16,), jnp.float32))
                s = _lane_sum(acc * wv, lanes)
                return jnp.where(lanes == r16, s, ovec)

            ovec = lax.fori_loop(0, 16, r_body, jnp.zeros((16,), jnp.float32))
            out_v[pl.ds(g * _GROUP_ROWS + q * 16, 16)] = ovec + bv
            return carry_q

        lax.fori_loop(0, _GROUP_ROWS // 16, q_body, 0)
        return carry

    lax.fori_loop(0, _GROUPS, group_body, 0)
    pltpu.sync_copy(out_v, out_hbm.at[pl.ds(wid * _ROWS_PER_W, _ROWS_PER_W)])


@jax.jit
def _run(text2, embedding, w_scaled, b_vec):
    mesh = plsc.VectorSubcoreMesh(core_axis_name="c", subcore_axis_name="s")
    k = pl.kernel(
        _sc_body,
        out_type=jax.ShapeDtypeStruct((_BATCH,), jnp.float32),
        mesh=mesh,
        scratch_types=[
            pltpu.VMEM((_GROUPS * _IDX_ROWS, _IDX_COLS), jnp.int32),
            pltpu.VMEM((_IDX_PER_GROUP, _D), jnp.float32),
            pltpu.VMEM((16,), jnp.float32),
            pltpu.VMEM((16,), jnp.float32),
            pltpu.VMEM((_ROWS_PER_W,), jnp.float32),
            pltpu.SemaphoreType.DMA,
        ],
        compiler_params=pltpu.CompilerParams(use_tc_tiling_on_sc=False),
    )
    return k(text2, embedding, w_scaled, b_vec)


def kernel(text, embedding, fc_w, fc_b):
    text2 = text.astype(jnp.int32).reshape(_IDX2_ROWS, _IDX_COLS)
    w_scaled = (fc_w[0] * (1.0 / _SEQ)).astype(jnp.float32)
    b_vec = jnp.broadcast_to(fc_b.astype(jnp.float32), (16,))
    return _run(text2, embedding, w_scaled, b_vec)


# speedup vs baseline: 1.7376x; 1.2131x over previous
import functools

import jax
import jax.numpy as jnp
from jax import lax
from jax.experimental import pallas as pl
from jax.experimental.pallas import tpu as pltpu, tpu_sc as plsc

_BATCH = 16384
_SEQ = 50
_D = 16
_NW = 32
_ROWS_PER_W = _BATCH // _NW
_GROUP_ROWS = 64
_GROUPS = _ROWS_PER_W // _GROUP_ROWS
_IDX_PER_GROUP = _GROUP_ROWS * _SEQ
_IDX_COLS = 128
_IDX_ROWS = _IDX_PER_GROUP // _IDX_COLS
_IDX2_ROWS = _BATCH * _SEQ // _IDX_COLS

_GATHER_DNUMS = lax.GatherDimensionNumbers(
    offset_dims=(), collapsed_slice_dims=(0,), start_index_map=(0,))


def _lane_shuffle(x, perm):
    return lax.gather(x, perm[:, None], _GATHER_DNUMS, (1,),
                      mode=lax.GatherScatterMode.PROMISE_IN_BOUNDS)


def _lane_sum(x, lanes):
    for sh in (8, 4, 2, 1):
        x = x + _lane_shuffle(x, lanes ^ sh)
    return x


def _sc_body(text_hbm, emb_hbm, w_hbm, b_hbm, out_hbm,
             idx_v, rows_v0, rows_v1, w_v, b_v, out_v, sem0, sem1):
    cid = lax.axis_index("c")
    sid = lax.axis_index("s")
    wid = cid * 16 + sid

    pltpu.sync_copy(w_hbm, w_v)
    pltpu.sync_copy(b_hbm, b_v)
    pltpu.sync_copy(
        text_hbm.at[pl.ds(wid * (_GROUPS * _IDX_ROWS), _GROUPS * _IDX_ROWS), :],
        idx_v)
    wv = w_v[...]
    bv = b_v[...]
    lanes = lax.iota(jnp.int32, 16)

    def fire(g, buf, sem):
        for j in range(_IDX_ROWS):
            pltpu.async_copy(
                emb_hbm.at[idx_v.at[g * _IDX_ROWS + j]],
                buf.at[pl.ds(j * _IDX_COLS, _IDX_COLS)],
                sem)

    def drain(buf, sem):
        pltpu.make_async_copy(
            emb_hbm.at[pl.ds(0, _IDX_PER_GROUP)], buf, sem).wait()

    def compute(g, buf):
        def q_body(q, carry_q):
            def r_body(r16, ovec):
                base = (q * 16 + r16) * _SEQ
                acc = buf[base, :]
                for l in range(1, _SEQ):
                    acc = acc + buf[base + l, :]
                s = _lane_sum(acc * wv, lanes)
                return jnp.where(lanes == r16, s, ovec)

            ovec = lax.fori_loop(0, 16, r_body, jnp.zeros((16,), jnp.float32))
            out_v[pl.ds(g * _GROUP_ROWS + q * 16, 16)] = ovec + bv
            return carry_q

        lax.fori_loop(0, _GROUP_ROWS // 16, q_body, 0)

    fire(0, rows_v0, sem0)

    def pipe_body(g4, carry):
        g = g4 * 2
        fire(g + 1, rows_v1, sem1)
        drain(rows_v0, sem0)
        compute(g, rows_v0)

        @pl.when(g + 2 < _GROUPS)
        def _():
            fire(g + 2, rows_v0, sem0)

        drain(rows_v1, sem1)
        compute(g + 1, rows_v1)
        return carry

    lax.fori_loop(0, _GROUPS // 2, pipe_body, 0)
    pltpu.sync_copy(out_v, out_hbm.at[pl.ds(wid * _ROWS_PER_W, _ROWS_PER_W)])


@jax.jit
def _run(text2, embedding, w_scaled, b_vec):
    mesh = plsc.VectorSubcoreMesh(core_axis_name="c", subcore_axis_name="s")
    k = pl.kernel(
        _sc_body,
        out_type=jax.ShapeDtypeStruct((_BATCH,), jnp.float32),
        mesh=mesh,
        scratch_types=[
            pltpu.VMEM((_GROUPS * _IDX_ROWS, _IDX_COLS), jnp.int32),
            pltpu.VMEM((_IDX_PER_GROUP, _D), jnp.float32),
            pltpu.VMEM((_IDX_PER_GROUP, _D), jnp.float32),
            pltpu.VMEM((16,), jnp.float32),
            pltpu.VMEM((16,), jnp.float32),
            pltpu.VMEM((_ROWS_PER_W,), jnp.float32),
            pltpu.SemaphoreType.DMA,
            pltpu.SemaphoreType.DMA,
        ],
        compiler_params=pltpu.CompilerParams(use_tc_tiling_on_sc=False),
    )
    return k(text2, embedding, w_scaled, b_vec)


def kernel(text, embedding, fc_w, fc_b):
    text2 = text.astype(jnp.int32).reshape(_IDX2_ROWS, _IDX_COLS)
    w_scaled = (fc_w[0] * (1.0 / _SEQ)).astype(jnp.float32)
    b_vec = jnp.broadcast_to(fc_b.astype(jnp.float32), (16,))
    return _run(text2, embedding, w_scaled, b_vec)
